# baseline (speedup 1.0000x reference)
import functools

import jax
import jax.numpy as jnp
from jax import lax
from jax.experimental import pallas as pl
from jax.experimental.pallas import tpu as pltpu
from jax.experimental.pallas import tpu_sc as plsc

N = 10000
E = 320000
D = 128

NC = 2
NS = 16
NW = NC * NS
C = 80
CHUNKS = 125
NP = 10240
RPT = NP // NS
ZB = 80
NBUF = 4
ISL = 4

_mesh = plsc.VectorSubcoreMesh(core_axis_name="c", subcore_axis_name="s")


@jax.jit
def _sc_spmm(x, adj4):
  @functools.partial(
      pl.kernel,
      out_type=jax.ShapeDtypeStruct((NC, NP, D), jnp.float32),
      mesh=_mesh,
      scratch_types=[
          pltpu.VMEM((ISL, 2, C), jnp.int32),
          pltpu.VMEM((NBUF, C, D), jnp.float32),
          pltpu.VMEM_SHARED((NP, D), jnp.float32),
          pltpu.SemaphoreType.DMA((NBUF,)),
          pltpu.SemaphoreType.DMA((NBUF,)),
          pltpu.SemaphoreType.DMA((ISL,)),
      ],
  )
  def sc_kernel(x_hbm, adj_hbm, out_hbm, idx_v, rows_v, t_sh, gsem, ssem,
                isem):
    c = lax.axis_index("c")
    s = lax.axis_index("s")
    wid = s * NC + c

    def stage(k, sl):
      return pltpu.make_async_copy(adj_hbm.at[wid, k], idx_v.at[sl],
                                   isem.at[sl])

    def gather(k, b, sl):
      return pltpu.make_async_copy(x_hbm.at[idx_v.at[sl, 0]], rows_v.at[b],
                                   gsem.at[b])

    def scatter(k, b, sl):
      return pltpu.make_async_copy(rows_v.at[b], t_sh.at[idx_v.at[sl, 1]],
                                   ssem.at[b])

    stage(0, 0).start()
    stage(1, 1).start()

    @pl.loop(0, ZB)
    def _(i):
      @pl.loop(0, D, step=16)
      def _(j):
        rows_v[0, i, pl.ds(j, 16)] = jnp.zeros((16,), jnp.float32)

    @pl.loop(0, RPT // ZB)
    def _(k):
      pltpu.sync_copy(rows_v.at[0], t_sh.at[pl.ds(s * RPT + k * ZB, ZB)])

    plsc.subcore_barrier()

    stage(0, 0).wait()
    gather(0, 0, 0).start()

    @pl.loop(0, CHUNKS - 1, step=ISL)
    def _(i):
      for b in range(ISL):
        k = i + b
        gather(k, b, b).wait()

        @pl.when(k > 1)
        def _():
          scatter(k - 2, (b - 2) % NBUF, (b - 2) % ISL).wait()

        scatter(k, b, b).start(add=True)

        @pl.when(k + 2 <= CHUNKS - 1)
        def _():
          stage(k + 2, (b + 2) % ISL).start()

        @pl.when(k + 1 <= CHUNKS - 1)
        def _():
          stage(k + 1, (b + 1) % ISL).wait()
          gather(k + 1, (b + 1) % NBUF, (b + 1) % ISL).start()

    gather(CHUNKS - 1, 0, 0).wait()
    scatter(CHUNKS - 3, 2, 2).wait()
    scatter(CHUNKS - 1, 0, 0).start(add=True)
    scatter(CHUNKS - 2, 3, 3).wait()
    scatter(CHUNKS - 1, 0, 0).wait()

    plsc.subcore_barrier()

    NWB = RPT // ZB

    def wb_load(k):
      return pltpu.make_async_copy(t_sh.at[pl.ds(s * RPT + k * ZB, ZB)],
                                   rows_v.at[k % NBUF], gsem.at[k % NBUF])

    def wb_store(k):
      return pltpu.make_async_copy(rows_v.at[k % NBUF],
                                   out_hbm.at[c, pl.ds(s * RPT + k * ZB, ZB)],
                                   ssem.at[k % NBUF])

    wb_load(0).start()
    for k in range(NWB):
      wb_load(k).wait()
      wb_store(k).start()
      if k > 0:
        wb_store(k - 1).wait()
      if k + 1 < NWB:
        wb_load(k + 1).start()
    wb_store(NWB - 1).wait()

  return sc_kernel(x, adj4)


def _mm_body(p_ref, w_ref, z_ref):
  t = p_ref[0] + p_ref[1]
  z_ref[...] = jnp.dot(t, w_ref[...], preferred_element_type=jnp.float32)


@jax.jit
def _mm(parts, weight):
  return pl.pallas_call(
      _mm_body,
      grid=(10,),
      in_specs=[
          pl.BlockSpec((2, N // 10, D), lambda i: (0, i, 0)),
          pl.BlockSpec((D, D), lambda i: (0, 0)),
      ],
      out_specs=pl.BlockSpec((N // 10, D), lambda i: (i, 0)),
      out_shape=jax.ShapeDtypeStruct((N, D), jnp.float32),
  )(parts, weight)


def kernel(x, adj, weight):
  adj4 = adj.reshape(2, NW, CHUNKS, C).transpose(1, 2, 0, 3)
  parts = _sc_spmm(x, adj4)
  return _mm(parts, weight)


# speedup vs baseline: 1.1651x; 1.1651x over previous
import functools

import jax
import jax.numpy as jnp
from jax import lax
from jax.experimental import pallas as pl
from jax.experimental.pallas import tpu as pltpu
from jax.experimental.pallas import tpu_sc as plsc

N = 10000
E = 320000
D = 128

NC = 2
NS = 16
NW = NC * NS
C = 125
CHUNKS = 80
NP = 10240
RPT = NP // NS
ZB = 80
NBUF = 2
ISL = 4

_mesh = plsc.VectorSubcoreMesh(core_axis_name="c", subcore_axis_name="s")


@jax.jit
def _sc_spmm(x, adj4):
  @functools.partial(
      pl.kernel,
      out_type=jax.ShapeDtypeStruct((NC, NP, D), jnp.float32),
      mesh=_mesh,
      scratch_types=[
          pltpu.VMEM((ISL, 2, C), jnp.int32),
          pltpu.VMEM((NBUF, C, D), jnp.float32),
          pltpu.VMEM_SHARED((NP, D), jnp.float32),
          pltpu.SemaphoreType.DMA((NBUF,)),
          pltpu.SemaphoreType.DMA((NBUF,)),
          pltpu.SemaphoreType.DMA((ISL,)),
      ],
  )
  def sc_kernel(x_hbm, adj_hbm, out_hbm, idx_v, rows_v, t_sh, gsem, ssem,
                isem):
    c = lax.axis_index("c")
    s = lax.axis_index("s")
    wid = s * NC + c

    def stage(k, sl):
      return pltpu.make_async_copy(adj_hbm.at[wid, k], idx_v.at[sl],
                                   isem.at[sl])

    def gather(k, b, sl):
      return pltpu.make_async_copy(x_hbm.at[idx_v.at[sl, 0]], rows_v.at[b],
                                   gsem.at[b])

    def scatter(k, b, sl):
      return pltpu.make_async_copy(rows_v.at[b], t_sh.at[idx_v.at[sl, 1]],
                                   ssem.at[b])

    stage(0, 0).start()
    stage(1, 1).start()

    @pl.loop(0, ZB)
    def _(i):
      @pl.loop(0, D, step=16)
      def _(j):
        rows_v[0, i, pl.ds(j, 16)] = jnp.zeros((16,), jnp.float32)

    @pl.loop(0, RPT // ZB)
    def _(k):
      pltpu.sync_copy(rows_v.at[0, pl.ds(0, ZB)],
                      t_sh.at[pl.ds(s * RPT + k * ZB, ZB)])

    plsc.subcore_barrier()

    stage(0, 0).wait()
    gather(0, 0, 0).start()

    @pl.loop(0, CHUNKS, step=ISL)
    def _(i):
      for b in range(ISL):
        k = i + b
        rb = b % NBUF
        gather(k, rb, b).wait()

        @pl.when(k > 0)
        def _():
          scatter(k - 1, (b - 1) % NBUF, (b - 1) % ISL).wait()

        scatter(k, rb, b).start(add=True)

        @pl.when(k + 2 <= CHUNKS - 1)
        def _():
          stage(k + 2, (b + 2) % ISL).start()

        @pl.when(k + 1 <= CHUNKS - 1)
        def _():
          stage(k + 1, (b + 1) % ISL).wait()
          gather(k + 1, (b + 1) % NBUF, (b + 1) % ISL).start()

    scatter(CHUNKS - 1, 1, 3).wait()

    plsc.subcore_barrier()

    NWB = RPT // ZB

    def wb_load(k):
      return pltpu.make_async_copy(t_sh.at[pl.ds(s * RPT + k * ZB, ZB)],
                                   rows_v.at[k % NBUF, pl.ds(0, ZB)],
                                   gsem.at[k % NBUF])

    def wb_store(k):
      return pltpu.make_async_copy(rows_v.at[k % NBUF, pl.ds(0, ZB)],
                                   out_hbm.at[c, pl.ds(s * RPT + k * ZB, ZB)],
                                   ssem.at[k % NBUF])

    wb_load(0).start()
    for k in range(NWB):
      wb_load(k).wait()
      wb_store(k).start()
      if k > 0:
        wb_store(k - 1).wait()
      if k + 1 < NWB:
        wb_load(k + 1).start()
    wb_store(NWB - 1).wait()

  return sc_kernel(x, adj4)


def _mm_body(p_ref, w_ref, z_ref):
  t = p_ref[0] + p_ref[1]
  z_ref[...] = jnp.dot(t, w_ref[...], preferred_element_type=jnp.float32)


@jax.jit
def _mm(parts, weight):
  return pl.pallas_call(
      _mm_body,
      grid=(10,),
      in_specs=[
          pl.BlockSpec((2, N // 10, D), lambda i: (0, i, 0)),
          pl.BlockSpec((D, D), lambda i: (0, 0)),
      ],
      out_specs=pl.BlockSpec((N // 10, D), lambda i: (i, 0)),
      out_shape=jax.ShapeDtypeStruct((N, D), jnp.float32),
  )(parts, weight)


def kernel(x, adj, weight):
  adj4 = adj.reshape(2, NW, CHUNKS, C).transpose(1, 2, 0, 3)
  parts = _sc_spmm(x, adj4)
  return _mm(parts, weight)
